# baseline (speedup 1.0000x reference)
import functools

import jax
import jax.numpy as jnp
from jax import lax
from jax.experimental import pallas as pl
from jax.experimental.pallas import tpu as pltpu
from jax.experimental.pallas import tpu_sc as plsc

CHUNK = 128
DP = 256


@functools.lru_cache(maxsize=None)
def _make_gather(B, V):
    info = plsc.get_sparse_core_info()
    NC, NS = info.num_cores, info.num_subcores
    NW = NC * NS
    assert B % (NW * CHUNK) == 0
    b_per_w = B // NW
    n_chunks = b_per_w // CHUNK
    assert n_chunks % 3 == 2
    mesh = plsc.VectorSubcoreMesh(core_axis_name="c", subcore_axis_name="s")

    @functools.partial(
        pl.kernel,
        mesh=mesh,
        out_type=jax.ShapeDtypeStruct((B, DP), jnp.float32),
        scratch_types=[
            pltpu.VMEM((b_per_w,), jnp.int32),
            pltpu.VMEM((CHUNK, DP), jnp.float32),
            pltpu.VMEM((CHUNK, DP), jnp.float32),
            pltpu.VMEM((CHUNK, DP), jnp.float32),
            pltpu.SemaphoreType.DMA,
            pltpu.SemaphoreType.DMA,
            pltpu.SemaphoreType.DMA,
            pltpu.SemaphoreType.DMA,
            pltpu.SemaphoreType.DMA,
            pltpu.SemaphoreType.DMA,
        ],
    )
    def gather_kernel(idx_hbm, table_hbm, out_hbm, idx_v, rows0, rows1,
                      rows2, gs0, gs1, gs2, ss0, ss1, ss2):
        wid = lax.axis_index("s") * NC + lax.axis_index("c")
        base = wid * b_per_w
        pltpu.sync_copy(idx_hbm.at[pl.ds(base, b_per_w)], idx_v)
        rows = (rows0, rows1, rows2)
        gsems = (gs0, gs1, gs2)
        ssems = (ss0, ss1, ss2)

        def start_gather(c, b):
            pltpu.async_copy(
                table_hbm.at[idx_v.at[pl.ds(c * CHUNK, CHUNK)]], rows[b],
                gsems[b])

        def wait_gather(c, b):
            pltpu.make_async_copy(
                table_hbm.at[idx_v.at[pl.ds(c * CHUNK, CHUNK)]], rows[b],
                gsems[b]).wait()

        def start_scatter(c, b):
            pltpu.async_copy(
                rows[b], out_hbm.at[pl.ds(base + c * CHUNK, CHUNK), :],
                ssems[b])

        def wait_scatter(c, b):
            pltpu.make_async_copy(
                rows[b], out_hbm.at[pl.ds(base + c * CHUNK, CHUNK), :],
                ssems[b]).wait()

        for r in range(3):
            start_gather(r, r)
        for r in range(3):
            wait_gather(r, r)
            start_scatter(r, r)

        def body(t, carry):
            c = 3 * t
            for r in range(3):
                wait_scatter(c + r - 3, r)
                start_gather(c + r, r)
            for r in range(3):
                wait_gather(c + r, r)
                start_scatter(c + r, r)
            return carry

        lax.fori_loop(1, (n_chunks - 2) // 3, body, 0)
        for r in range(2):
            c = n_chunks - 2 + r
            wait_scatter(c - 3, r)
            start_gather(c, r)
        for r in range(2):
            c = n_chunks - 2 + r
            wait_gather(c, r)
            start_scatter(c, r)
        wait_scatter(n_chunks - 3, 2)
        wait_scatter(n_chunks - 2, 0)
        wait_scatter(n_chunks - 1, 1)

    return gather_kernel


def kernel(x, table):
    B, S = x.shape
    V, D = table.shape
    flat = x.reshape(B * S).astype(jnp.int32)
    table_p = jnp.pad(table, ((0, 0), (0, DP - D)))
    out = _make_gather(B * S, V)(flat, table_p)
    return out[:, :D].reshape(B, S, D)
